# baseline (speedup 1.0000x reference)
import functools

import jax
import jax.numpy as jnp
import numpy as np
from jax import lax
from jax.experimental import pallas as pl
from jax.experimental.pallas import tpu as pltpu

_N = 5000
_NP = 5120
_T = 256
_NT = _NP // _T
_SIGMA = 0.5

_PAIRS = np.array([(a, b) for a in range(_NT) for b in range(a, _NT)],
                  dtype=np.int32).T.copy()
_NUM_TILES = _PAIRS.shape[1]


def _decode_rows(rows):
    cx = rows[:, 0:1] * 100.0
    cy = rows[:, 1:2] * 100.0
    w = rows[:, 2:3] * 10.0 + 1e-3
    h = rows[:, 3:4] * 10.0 + 1e-3
    return (cx - w * 0.5, cx + w * 0.5, cy - h * 0.5, cy + h * 0.5,
            w * h, rows[:, 4:5])


def _decode_cols(cols):
    cx = cols[0:1, :] * 100.0
    cy = cols[1:2, :] * 100.0
    w = cols[2:3, :] * 10.0 + 1e-3
    h = cols[3:4, :] * 10.0 + 1e-3
    return (cx - w * 0.5, cx + w * 0.5, cy - h * 0.5, cy + h * 0.5,
            w * h, cols[4:5, :])


def _tile_kernel(tab_ref, rows_ref, cols_ref, accr_ref, accc_ref):
    t = pl.program_id(0)
    a = tab_ref[0, t]
    b = tab_ref[1, t]

    @pl.when(t == 0)
    def _init():
        accr_ref[...] = jnp.zeros((_NP, 128), jnp.float32)
        accc_ref[...] = jnp.zeros((8, _NP), jnp.float32)

    rows = rows_ref[pl.ds(a * _T, _T), :]
    cols = cols_ref[:, pl.ds(b * _T, _T)]
    x1r, x2r, y1r, y2r, ar, sr = _decode_rows(rows)
    x1c, x2c, y1c, y2c, ac, sc = _decode_cols(cols)

    iw = jnp.maximum(jnp.minimum(x2r, x2c) - jnp.maximum(x1r, x1c), 0.0)
    ih = jnp.maximum(jnp.minimum(y2r, y2c) - jnp.maximum(y1r, y1c), 0.0)
    inter = iw * ih
    iou = inter / (ar + ac - inter + 1e-8)
    iou2 = iou * iou

    rmask = sc > sr
    rsum = jnp.sum(jnp.where(rmask, iou2, 0.0).reshape(_T, _T // 128, 128),
                   axis=1)
    accr_ref[pl.ds(a * _T, _T), :] += rsum

    @pl.when(b != a)
    def _cols():
        csum = jnp.sum(jnp.where(sr > sc, iou2, 0.0).reshape(_T // 8, 8, _T),
                       axis=0)
        accc_ref[:, pl.ds(b * _T, _T)] += csum


def _final_kernel(accr_ref, accct_ref, s_ref, out_ref):
    total = (jnp.sum(accr_ref[...], axis=1, keepdims=True)
             + jnp.sum(accct_ref[...], axis=1, keepdims=True))
    new = s_ref[...] * jnp.exp(-total / _SIGMA)
    out_ref[...] = jnp.where(new > 0.1, new, 0.0)


@jax.jit
def kernel(boxes, scores):
    feats = jnp.zeros((_NP, 8), jnp.float32)
    feats = feats.at[:_N, 0:4].set(boxes)
    spad = jnp.pad(scores, (0, _NP - _N), constant_values=-1.0)
    feats = feats.at[:, 4].set(spad)
    cols = feats.T

    accr, accc = pl.pallas_call(
        _tile_kernel,
        grid_spec=pltpu.PrefetchScalarGridSpec(
            num_scalar_prefetch=1,
            grid=(_NUM_TILES,),
            in_specs=[
                pl.BlockSpec((_NP, 8), lambda t, tab: (0, 0)),
                pl.BlockSpec((8, _NP), lambda t, tab: (0, 0)),
            ],
            out_specs=[
                pl.BlockSpec((_NP, 128), lambda t, tab: (0, 0)),
                pl.BlockSpec((8, _NP), lambda t, tab: (0, 0)),
            ],
        ),
        out_shape=[
            jax.ShapeDtypeStruct((_NP, 128), jnp.float32),
            jax.ShapeDtypeStruct((8, _NP), jnp.float32),
        ],
    )(jnp.asarray(_PAIRS), feats, cols)

    out = pl.pallas_call(
        _final_kernel,
        out_shape=jax.ShapeDtypeStruct((_NP, 1), jnp.float32),
    )(accr, accc.T, spad.reshape(_NP, 1))
    return out[:_N, 0]


# speedup vs baseline: 2.2483x; 2.2483x over previous
import functools

import jax
import jax.numpy as jnp
import numpy as np
from jax import lax
from jax.experimental import pallas as pl
from jax.experimental.pallas import tpu as pltpu
from jax.experimental.pallas import tpu_sc as plsc

_N = 5000
_NP = 5120
_T = 512
_NT = _NP // _T
_SIGMA = 0.5
_D = 128

_PAIRS = np.array([(a, b) for a in range(_NT) for b in range(a, _NT)],
                  dtype=np.int32).T.copy()
_NUM_TILES = _PAIRS.shape[1]

_SC_INFO = plsc.get_sparse_core_info()
_NW = _SC_INFO.num_cores * _SC_INFO.num_subcores
_BPW = _NP // _NW
_SC_MESH = plsc.VectorSubcoreMesh(core_axis_name="c", subcore_axis_name="s")


@functools.partial(
    pl.kernel, mesh=_SC_MESH,
    out_type=jax.ShapeDtypeStruct((_NP, _D), jnp.float32),
    scratch_types=[
        pltpu.VMEM((_BPW,), jnp.int32),
        pltpu.VMEM((_BPW, _D), jnp.float32),
        pltpu.SemaphoreType.DMA,
    ],
)
def _sc_gather(table_hbm, idx_hbm, out_hbm, idx_v, rows_v, sem):
    wid = lax.axis_index("s") * _SC_INFO.num_cores + lax.axis_index("c")
    base = wid * _BPW
    pltpu.sync_copy(idx_hbm.at[pl.ds(base, _BPW)], idx_v)
    pltpu.async_copy(table_hbm.at[idx_v], rows_v, sem).wait()
    pltpu.sync_copy(rows_v, out_hbm.at[pl.ds(base, _BPW)])


@functools.partial(
    pl.kernel, mesh=_SC_MESH,
    out_type=jax.ShapeDtypeStruct((_NP, _D), jnp.float32),
    scratch_types=[
        pltpu.VMEM((_BPW,), jnp.int32),
        pltpu.VMEM((_BPW, _D), jnp.float32),
        pltpu.SemaphoreType.DMA,
    ],
)
def _sc_scatter(vals_hbm, idx_hbm, out_hbm, idx_v, rows_v, sem):
    wid = lax.axis_index("s") * _SC_INFO.num_cores + lax.axis_index("c")
    base = wid * _BPW
    pltpu.sync_copy(idx_hbm.at[pl.ds(base, _BPW)], idx_v)
    pltpu.sync_copy(vals_hbm.at[pl.ds(base, _BPW)], rows_v)
    pltpu.async_copy(rows_v, out_hbm.at[idx_v], sem).wait()


def _decode_rows(rows):
    cx = rows[:, 0:1] * 100.0
    cy = rows[:, 1:2] * 100.0
    w = rows[:, 2:3] * 10.0 + 1e-3
    h = rows[:, 3:4] * 10.0 + 1e-3
    return (cx - w * 0.5, cx + w * 0.5, cy - h * 0.5, cy + h * 0.5,
            w * h, rows[:, 4:5])


def _decode_cols(cols):
    cx = cols[0:1, :] * 100.0
    cy = cols[1:2, :] * 100.0
    w = cols[2:3, :] * 10.0 + 1e-3
    h = cols[3:4, :] * 10.0 + 1e-3
    return (cx - w * 0.5, cx + w * 0.5, cy - h * 0.5, cy + h * 0.5,
            w * h + 1e-8, cols[4:5, :])


def _tile_kernel(tab_ref, rows_ref, cols_ref, accr_ref, accc_ref,
                 maxx2_ref, minx1_ref):
    t = pl.program_id(0)
    a = tab_ref[0, t]
    b = tab_ref[1, t]

    @pl.when(t == 0)
    def _init():
        accr_ref[...] = jnp.zeros((_NP, 128), jnp.float32)
        accc_ref[...] = jnp.zeros((8, _NP), jnp.float32)
        allr = rows_ref[...]
        cx = allr[:, 0:1] * 100.0
        wd = allr[:, 2:3] * 10.0 + 1e-3
        x1 = cx - wd * 0.5
        x2 = cx + wd * 0.5
        for blk in range(_NT):
            maxx2_ref[blk] = jnp.max(x2[blk * _T:(blk + 1) * _T, 0])
            minx1_ref[blk] = jnp.min(x1[blk * _T:(blk + 1) * _T, 0])

    live = jnp.logical_or(a == b, minx1_ref[b] <= maxx2_ref[a])

    @pl.when(live)
    def _compute():
        rows = rows_ref[pl.ds(a * _T, _T), :]
        cols = cols_ref[:, pl.ds(b * _T, _T)]
        x1r, x2r, y1r, y2r, ar, sr = _decode_rows(rows)
        x1c, x2c, y1c, y2c, ac, sc = _decode_cols(cols)

        iw = jnp.maximum(jnp.minimum(x2r, x2c) - jnp.maximum(x1r, x1c), 0.0)
        ih = jnp.maximum(jnp.minimum(y2r, y2c) - jnp.maximum(y1r, y1c), 0.0)
        inter = iw * ih
        iou = inter / ((ar + ac) - inter)
        iou2 = iou * iou

        rsel = jnp.where(sc > sr, iou2, 0.0)
        w = _T
        while w > 128:
            w //= 2
            rsel = rsel[:, :w] + rsel[:, w:]
        accr_ref[pl.ds(a * _T, _T), :] += rsel

        @pl.when(b != a)
        def _cols():
            csel = jnp.where(sr > sc, iou2, 0.0)
            hgt = _T
            while hgt > 8:
                hgt //= 2
                csel = csel[:hgt, :] + csel[hgt:, :]
            accc_ref[:, pl.ds(b * _T, _T)] += csel


def _final_kernel(accr_ref, accct_ref, s_ref, out_ref):
    total = (jnp.sum(accr_ref[...], axis=1, keepdims=True)
             + jnp.sum(accct_ref[...], axis=1, keepdims=True))
    new = s_ref[...] * jnp.exp(-total / _SIGMA)
    out_ref[...] = jnp.broadcast_to(jnp.where(new > 0.1, new, 0.0), (_NP, _D))


@jax.jit
def kernel(boxes, scores):
    cxbits = lax.bitcast_convert_type(boxes[:, 0], jnp.uint32)
    key = (cxbits & np.uint32(0xFFFFE000)) | lax.iota(jnp.uint32, _N)
    order = (jnp.sort(key) & np.uint32(0x1FFF)).astype(jnp.int32)
    order_full = jnp.concatenate([order, lax.iota(jnp.int32, _NP - _N) + _N])

    pw = (0, _NP - _N)
    feats0 = jnp.pad(jnp.stack(
        [jnp.pad(boxes[:, 0], pw, constant_values=1e4),
         jnp.pad(boxes[:, 1], pw), jnp.pad(boxes[:, 2], pw),
         jnp.pad(boxes[:, 3], pw),
         jnp.pad(scores, pw, constant_values=-1.0)],
        axis=1), ((0, 0), (0, _D - 5)))

    feats = _sc_gather(feats0, order_full)
    spad = feats[:, 4]
    cols = feats[:, :8].T

    accr, accc = pl.pallas_call(
        _tile_kernel,
        grid_spec=pltpu.PrefetchScalarGridSpec(
            num_scalar_prefetch=1,
            grid=(_NUM_TILES,),
            in_specs=[
                pl.BlockSpec((_NP, _D), lambda t, tab: (0, 0)),
                pl.BlockSpec((8, _NP), lambda t, tab: (0, 0)),
            ],
            out_specs=[
                pl.BlockSpec((_NP, 128), lambda t, tab: (0, 0)),
                pl.BlockSpec((8, _NP), lambda t, tab: (0, 0)),
            ],
            scratch_shapes=[
                pltpu.SMEM((_NT,), jnp.float32),
                pltpu.SMEM((_NT,), jnp.float32),
            ],
        ),
        out_shape=[
            jax.ShapeDtypeStruct((_NP, 128), jnp.float32),
            jax.ShapeDtypeStruct((8, _NP), jnp.float32),
        ],
    )(jnp.asarray(_PAIRS), feats, cols)

    out_sorted = pl.pallas_call(
        _final_kernel,
        out_shape=jax.ShapeDtypeStruct((_NP, _D), jnp.float32),
    )(accr, accc.T, spad.reshape(_NP, 1))

    out = _sc_scatter(out_sorted, order_full)
    return out[:_N, 0]
